# baseline (speedup 1.0000x reference)
import functools

import jax
import jax.numpy as jnp
from jax import lax
from jax.experimental import pallas as pl
from jax.experimental.pallas import tpu as pltpu
from jax.experimental.pallas import tpu_sc as plsc

DIM = 1024
ROWS = 16384
NC, NS, L = 2, 16, 16
NW = NC * NS
B_PER_W = ROWS // NW
CH = 32
N_CHUNKS = B_PER_W // CH


def _body(pos0_hbm, pos1_hbm, pe0_hbm, pe1_hbm, out_hbm,
          idx0_v, idx1_v, buf_a, buf_b, sem_a, sem_b):
    wid = lax.axis_index("s") * NC + lax.axis_index("c")
    base = wid * B_PER_W
    pltpu.sync_copy(pos0_hbm.at[pl.ds(base, B_PER_W)], idx0_v)
    pltpu.sync_copy(pos1_hbm.at[pl.ds(base, B_PER_W)], idx1_v)

    for c in range(N_CHUNKS):
        cpy_a = pltpu.make_async_copy(
            pe0_hbm.at[idx0_v.at[pl.ds(c * CH, CH)]], buf_a, sem_a)
        cpy_b = pltpu.make_async_copy(
            pe1_hbm.at[idx1_v.at[pl.ds(c * CH, CH)]], buf_b, sem_b)
        cpy_a.start()
        cpy_b.start()
        cpy_a.wait()
        cpy_b.wait()

        def add_row(r, _):
            def add_vec(j, _):
                buf_a[r, pl.ds(j * L, L)] = (
                    buf_a[r, pl.ds(j * L, L)] + buf_b[r, pl.ds(j * L, L)])
                return 0
            lax.fori_loop(0, DIM // L, add_vec, 0)
            return 0
        lax.fori_loop(0, CH, add_row, 0)

        pltpu.sync_copy(buf_a, out_hbm.at[pl.ds(base + c * CH, CH)])


@jax.jit
def _run(pos0f, pos1f, pe0, pe1):
    mesh = plsc.VectorSubcoreMesh(core_axis_name="c", subcore_axis_name="s")
    f = pl.kernel(
        _body,
        out_type=jax.ShapeDtypeStruct((ROWS, DIM), jnp.float32),
        mesh=mesh,
        scratch_types=[
            pltpu.VMEM((B_PER_W,), jnp.int32),
            pltpu.VMEM((B_PER_W,), jnp.int32),
            pltpu.VMEM((CH, DIM), jnp.float32),
            pltpu.VMEM((CH, DIM), jnp.float32),
            pltpu.SemaphoreType.DMA,
            pltpu.SemaphoreType.DMA,
        ],
    )
    return f(pos0f, pos1f, pe0, pe1)


def kernel(pos0, pos1, pe0, pe1):
    batch, seq = pos0.shape
    pos0f = pos0.reshape(-1).astype(jnp.int32)
    pos1f = pos1.reshape(-1).astype(jnp.int32)
    out = _run(pos0f, pos1f, pe0, pe1)
    return out.reshape(batch, seq, DIM)


# speedup vs baseline: 1.5757x; 1.5757x over previous
import jax
import jax.numpy as jnp
from jax import lax
from jax.experimental import pallas as pl
from jax.experimental.pallas import tpu as pltpu
from jax.experimental.pallas import tpu_sc as plsc

DIM = 1024
ROWS = 16384
NC, NS, L = 2, 16, 16
NW = NC * NS
B_PER_W = ROWS // NW
CH = 16
N_CHUNKS = B_PER_W // CH


def _body(pos0_hbm, pos1_hbm, pe0_hbm, pe1_hbm, out_hbm,
          idx0_v, idx1_v, a0, a1, b0, b1, sg0, sg1, so0, so1):
    wid = lax.axis_index("s") * NC + lax.axis_index("c")
    base = wid * B_PER_W
    pltpu.sync_copy(pos0_hbm.at[pl.ds(base, B_PER_W)], idx0_v)
    pltpu.sync_copy(pos1_hbm.at[pl.ds(base, B_PER_W)], idx1_v)

    bufs = ((a0, b0, sg0, so0), (a1, b1, sg1, so1))

    def start_gather(c, slot):
        a, b, sg, _ = bufs[slot]
        off = pl.multiple_of(c * CH, CH)
        pltpu.make_async_copy(pe0_hbm.at[idx0_v.at[pl.ds(off, CH)]], a, sg).start()
        pltpu.make_async_copy(pe1_hbm.at[idx1_v.at[pl.ds(off, CH)]], b, sg).start()

    def wait_gather(slot):
        a, b, sg, _ = bufs[slot]
        pltpu.make_async_copy(pe0_hbm.at[idx0_v.at[pl.ds(0, CH)]], a, sg).wait()
        pltpu.make_async_copy(pe1_hbm.at[idx1_v.at[pl.ds(0, CH)]], b, sg).wait()

    def start_scatter(c, slot):
        a, _, _, so = bufs[slot]
        off = pl.multiple_of(base + c * CH, CH)
        pltpu.make_async_copy(a, out_hbm.at[pl.ds(off, CH)], so).start()

    def wait_scatter(slot):
        a, _, _, so = bufs[slot]
        pltpu.make_async_copy(a, out_hbm.at[pl.ds(0, CH)], so).wait()

    def add_chunk(a, b):
        def add_row(r, _):
            for j in range(DIM // L):
                sl = pl.ds(j * L, L)
                plsc.addupdate(a.at[r, sl], b[r, sl])
            return 0
        lax.fori_loop(0, CH, add_row, 0, unroll=False)

    start_gather(0, 0)

    def group(g, _):
        for slot in (0, 1):
            c = g * 2 + slot
            a, b, _, _ = bufs[slot]

            @pl.when(c >= 1)
            def _():
                wait_scatter(1 - slot)

            @pl.when(c + 1 < N_CHUNKS)
            def _():
                start_gather(c + 1, 1 - slot)

            wait_gather(slot)
            add_chunk(a, b)
            start_scatter(c, slot)
        return 0

    lax.fori_loop(0, N_CHUNKS // 2, group, 0, unroll=False)
    wait_scatter(1)


@jax.jit
def _run(pos0f, pos1f, pe0, pe1):
    mesh = plsc.VectorSubcoreMesh(core_axis_name="c", subcore_axis_name="s")
    f = pl.kernel(
        _body,
        out_type=jax.ShapeDtypeStruct((ROWS, DIM), jnp.float32),
        mesh=mesh,
        scratch_types=[
            pltpu.VMEM((B_PER_W,), jnp.int32),
            pltpu.VMEM((B_PER_W,), jnp.int32),
            pltpu.VMEM((CH, DIM), jnp.float32),
            pltpu.VMEM((CH, DIM), jnp.float32),
            pltpu.VMEM((CH, DIM), jnp.float32),
            pltpu.VMEM((CH, DIM), jnp.float32),
            pltpu.SemaphoreType.DMA,
            pltpu.SemaphoreType.DMA,
            pltpu.SemaphoreType.DMA,
            pltpu.SemaphoreType.DMA,
        ],
    )
    return f(pos0f, pos1f, pe0, pe1)


def kernel(pos0, pos1, pe0, pe1):
    batch, seq = pos0.shape
    pos0f = pos0.reshape(-1).astype(jnp.int32)
    pos1f = pos1.reshape(-1).astype(jnp.int32)
    out = _run(pos0f, pos1f, pe0, pe1)
    return out.reshape(batch, seq, DIM)


# speedup vs baseline: 1.7121x; 1.0866x over previous
import jax
import jax.numpy as jnp
from jax import lax
from jax.experimental import pallas as pl
from jax.experimental.pallas import tpu as pltpu
from jax.experimental.pallas import tpu_sc as plsc

DIM = 1024
ROWS = 16384
NC, NS, L = 2, 16, 16
NW = NC * NS
B_PER_W = ROWS // NW
CH = 16
N_CHUNKS = B_PER_W // CH
NSLOT = 3
N_MAIN = (N_CHUNKS // NSLOT) * NSLOT


def _body(pos0_hbm, pos1_hbm, pe0_hbm, pe1_hbm, out_hbm,
          idx0_v, idx1_v, a_bufs, b_bufs, sg, so):
    wid = lax.axis_index("s") * NC + lax.axis_index("c")
    base = wid * B_PER_W
    pltpu.sync_copy(pos0_hbm.at[pl.ds(base, B_PER_W)], idx0_v)
    pltpu.sync_copy(pos1_hbm.at[pl.ds(base, B_PER_W)], idx1_v)

    def start_gather(c, slot):
        off = pl.multiple_of(c * CH, CH)
        pltpu.make_async_copy(
            pe0_hbm.at[idx0_v.at[pl.ds(off, CH)]], a_bufs[slot], sg[slot]).start()
        pltpu.make_async_copy(
            pe1_hbm.at[idx1_v.at[pl.ds(off, CH)]], b_bufs[slot], sg[slot]).start()

    def wait_gather(slot):
        pltpu.make_async_copy(
            pe0_hbm.at[idx0_v.at[pl.ds(0, CH)]], a_bufs[slot], sg[slot]).wait()
        pltpu.make_async_copy(
            pe1_hbm.at[idx1_v.at[pl.ds(0, CH)]], b_bufs[slot], sg[slot]).wait()

    def start_scatter(c, slot):
        off = pl.multiple_of(base + c * CH, CH)
        pltpu.make_async_copy(
            a_bufs[slot], out_hbm.at[pl.ds(off, CH)], so[slot]).start()

    def wait_scatter(slot):
        pltpu.make_async_copy(
            a_bufs[slot], out_hbm.at[pl.ds(0, CH)], so[slot]).wait()

    def add_chunk(slot):
        a, b = a_bufs[slot], b_bufs[slot]

        def add_row(r, _):
            for j in range(DIM // L):
                sl = pl.ds(j * L, L)
                plsc.addupdate(a.at[r, sl], b[r, sl])
            return 0
        lax.fori_loop(0, CH, add_row, 0, unroll=False)

    def step(c, slot):
        @pl.when(c >= 2)
        def _():
            wait_scatter((slot + 1) % NSLOT)

        @pl.when(c + 1 < N_CHUNKS)
        def _():
            start_gather(c + 1, (slot + 1) % NSLOT)

        wait_gather(slot)
        add_chunk(slot)
        start_scatter(c, slot)

    start_gather(0, 0)

    def group(g, _):
        for t in range(NSLOT):
            step(g * NSLOT + t, t)
        return 0

    lax.fori_loop(0, N_MAIN // NSLOT, group, 0, unroll=False)

    for c in range(N_MAIN, N_CHUNKS):
        step(jnp.int32(c), c % NSLOT)

    for c in range(N_CHUNKS - 2, N_CHUNKS):
        wait_scatter(c % NSLOT)


@jax.jit
def _run(pos0f, pos1f, pe0, pe1):
    mesh = plsc.VectorSubcoreMesh(core_axis_name="c", subcore_axis_name="s")

    def body(pos0r, pos1r, pe0r, pe1r, outr,
             idx0_v, idx1_v, a0, a1, a2, b0, b1, b2,
             sg0, sg1, sg2, so0, so1, so2):
        _body(pos0r, pos1r, pe0r, pe1r, outr, idx0_v, idx1_v,
              (a0, a1, a2), (b0, b1, b2), (sg0, sg1, sg2), (so0, so1, so2))

    f = pl.kernel(
        body,
        out_type=jax.ShapeDtypeStruct((ROWS, DIM), jnp.float32),
        mesh=mesh,
        scratch_types=(
            [pltpu.VMEM((B_PER_W,), jnp.int32)] * 2
            + [pltpu.VMEM((CH, DIM), jnp.float32)] * 6
            + [pltpu.SemaphoreType.DMA] * 6
        ),
    )
    return f(pos0f, pos1f, pe0, pe1)


def kernel(pos0, pos1, pe0, pe1):
    batch, seq = pos0.shape
    pos0f = pos0.reshape(-1).astype(jnp.int32)
    pos1f = pos1.reshape(-1).astype(jnp.int32)
    out = _run(pos0f, pos1f, pe0, pe1)
    return out.reshape(batch, seq, DIM)


# speedup vs baseline: 2.7031x; 1.5788x over previous
import jax
import jax.numpy as jnp
from jax import lax
from jax.experimental import pallas as pl
from jax.experimental.pallas import tpu as pltpu
from jax.experimental.pallas import tpu_sc as plsc

DIM = 1024
ROWS = 16384
NC, NS, L = 2, 16, 16
NW = NC * NS
B_PER_W = ROWS // NW
CH = 8
N_CHUNKS = B_PER_W // CH
NSLOT = 4
PRE = 2


def _body(pos0_hbm, pos1_hbm, pe0_hbm, pe1_hbm, out_hbm,
          idx0_v, idx1_v, a_bufs, b_bufs, sg, so):
    wid = lax.axis_index("s") * NC + lax.axis_index("c")
    base = wid * B_PER_W
    pltpu.sync_copy(pos0_hbm.at[pl.ds(base, B_PER_W)], idx0_v)
    pltpu.sync_copy(pos1_hbm.at[pl.ds(base, B_PER_W)], idx1_v)

    def start_gather(c, slot):
        off = pl.multiple_of(c * CH, CH)
        pltpu.make_async_copy(
            pe0_hbm.at[idx0_v.at[pl.ds(off, CH)]], a_bufs[slot], sg[slot]).start()
        pltpu.make_async_copy(
            pe1_hbm.at[idx1_v.at[pl.ds(off, CH)]], b_bufs[slot], sg[slot]).start()

    def wait_gather(slot):
        pltpu.make_async_copy(
            pe0_hbm.at[idx0_v.at[pl.ds(0, CH)]], a_bufs[slot], sg[slot]).wait()
        pltpu.make_async_copy(
            pe1_hbm.at[idx1_v.at[pl.ds(0, CH)]], b_bufs[slot], sg[slot]).wait()

    def start_scatter(c, slot):
        off = pl.multiple_of(base + c * CH, CH)
        pltpu.make_async_copy(
            a_bufs[slot], out_hbm.at[pl.ds(off, CH)], so[slot]).start()

    def wait_scatter(slot):
        pltpu.make_async_copy(
            a_bufs[slot], out_hbm.at[pl.ds(0, CH)], so[slot]).wait()

    def add_chunk(slot):
        a, b = a_bufs[slot], b_bufs[slot]

        def add_row(r, _):
            for j in range(DIM // L):
                sl = pl.ds(j * L, L)
                plsc.addupdate(a.at[r, sl], b[r, sl])
            return 0
        lax.fori_loop(0, CH, add_row, 0, unroll=False)

    def step(c, slot):
        @pl.when(c >= NSLOT - PRE)
        def _():
            wait_scatter((slot + PRE) % NSLOT)

        @pl.when(c + PRE < N_CHUNKS)
        def _():
            start_gather(c + PRE, (slot + PRE) % NSLOT)

        wait_gather(slot)
        add_chunk(slot)
        start_scatter(c, slot)

    for p in range(PRE):
        start_gather(p, p)

    def group(g, _):
        for t in range(NSLOT):
            step(g * NSLOT + t, t)
        return 0

    lax.fori_loop(0, N_CHUNKS // NSLOT, group, 0, unroll=False)

    for c in range(N_CHUNKS - (NSLOT - PRE), N_CHUNKS):
        wait_scatter(c % NSLOT)


@jax.jit
def _run(pos0f, pos1f, pe0, pe1):
    mesh = plsc.VectorSubcoreMesh(core_axis_name="c", subcore_axis_name="s")

    def body(pos0r, pos1r, pe0r, pe1r, outr, idx0_v, idx1_v, *rest):
        a_bufs = rest[0:NSLOT]
        b_bufs = rest[NSLOT:2 * NSLOT]
        sg = rest[2 * NSLOT:3 * NSLOT]
        so = rest[3 * NSLOT:4 * NSLOT]
        _body(pos0r, pos1r, pe0r, pe1r, outr, idx0_v, idx1_v,
              a_bufs, b_bufs, sg, so)

    f = pl.kernel(
        body,
        out_type=jax.ShapeDtypeStruct((ROWS, DIM), jnp.float32),
        mesh=mesh,
        scratch_types=(
            [pltpu.VMEM((B_PER_W,), jnp.int32)] * 2
            + [pltpu.VMEM((CH, DIM), jnp.float32)] * (2 * NSLOT)
            + [pltpu.SemaphoreType.DMA] * (2 * NSLOT)
        ),
    )
    return f(pos0f, pos1f, pe0, pe1)


def kernel(pos0, pos1, pe0, pe1):
    batch, seq = pos0.shape
    pos0f = pos0.reshape(-1).astype(jnp.int32)
    pos1f = pos1.reshape(-1).astype(jnp.int32)
    out = _run(pos0f, pos1f, pe0, pe1)
    return out.reshape(batch, seq, DIM)
